# baseline (speedup 1.0000x reference)
import functools

import jax
import jax.numpy as jnp
from jax import lax
from jax.experimental import pallas as pl
from jax.experimental.pallas import tpu as pltpu
from jax.experimental.pallas import tpu_sc as plsc

N = 64
R = 64
D = 64 * 64
NC = 2
NS = 16
L = 16
NW = NC * NS
D_PER_W = N // NW
NB = 8
NBATCH = N // NB
NBUF = 2


def _sc_body(x_hbm, y_hbm, a_hbm, out_hbm,
             x_v, y_v, w0_v, w1_v, idx_v, rows_v, out_v,
             sem_g, sem_o0, sem_o1, sem_o2):
    c = lax.axis_index("c")
    s = lax.axis_index("s")
    wid = s * NC + c
    d_base = wid * D_PER_W

    pltpu.sync_copy(x_hbm, x_v)
    pltpu.sync_copy(y_hbm, y_v)

    lanes = lax.iota(jnp.int32, L)

    samp = jnp.minimum(d_base + (lanes >> 1), N - 1)
    yv = plsc.load_gather(y_v, [samp])
    ty = yv * 2.0 - 1.0
    yy = 0.5 * ((ty + 1.0) * 62.0)
    y0i = yy.astype(jnp.int32)
    y0c = jnp.clip(y0i, 0, R - 1)
    y1c = jnp.clip(y0i + 1, 0, R - 1)
    rowidx = jnp.where((lanes & 1) == 0, y0c, y1c)
    plsc.store_scatter(idx_v, [lanes & 3], rowidx, mask=lanes < 2 * D_PER_W)

    gather = pltpu.async_copy(a_hbm.at[idx_v], rows_v, sem_g)

    for k in range(N // L):
        sl = pl.ds(k * L, L)
        xk = x_v[sl]
        yk = y_v[sl]
        txk = xk * 2.0 - 1.0
        xx = 0.5 * ((txk + 1.0) * 62.0)
        x0i = xx.astype(jnp.int32)
        x0f = jnp.clip(x0i, 0, R - 1).astype(jnp.float32)
        x1f = jnp.clip(x0i + 1, 0, R - 1).astype(jnp.float32)
        xfac = (x1f - xx) + (xx - x0f)
        tyk = yk * 2.0 - 1.0
        yyk = 0.5 * ((tyk + 1.0) * 62.0)
        ky0 = yyk.astype(jnp.int32)
        ky0f = jnp.clip(ky0, 0, R - 1).astype(jnp.float32)
        ky1f = jnp.clip(ky0 + 1, 0, R - 1).astype(jnp.float32)
        sl_w = pl.ds((k + 1) * L, L)
        w0_v[sl_w] = xfac * (ky1f - yyk)
        w1_v[sl_w] = xfac * (yyk - ky0f)

    gather.wait()

    out_handles = [[] for _ in range(NBUF)]
    out_sems = [sem_o0, sem_o1, sem_o2][:NBUF]
    for di in range(D_PER_W):
        d = d_base + di
        for nb in range(NBATCH):
            b = (di * NBATCH + nb) % NBUF
            w0b = [plsc.load_gather(w0_v, [jnp.full((L,), L + nb * NB + k,
                                                    jnp.int32)])
                   for k in range(NB)]
            w1b = [plsc.load_gather(w1_v, [jnp.full((L,), L + nb * NB + k,
                                                    jnp.int32)])
                   for k in range(NB)]
            for h in out_handles[b]:
                h.wait()
            out_handles[b] = []

            def chunk(j, di=di, b=b, w0b=w0b, w1b=w1b):
                sl = pl.ds(j * L, L)
                r0 = rows_v[2 * di, sl]
                r1 = rows_v[2 * di + 1, sl]
                for k in range(NB):
                    out_v[b, k, sl] = w0b[k] * r0 + w1b[k] * r1

            plsc.parallel_loop(0, D // L, 1, unroll=8)(chunk)
            out_handles[b] = [
                pltpu.async_copy(
                    out_v.at[b, k],
                    out_hbm.at[pl.ds((d * N + nb * NB + k) * D, D)],
                    out_sems[b])
                for k in range(NB)]
    for hs in out_handles:
        for h in hs:
            h.wait()


@jax.jit
def _alpha_grid(x_col, y_col, a2d):
    mesh = plsc.VectorSubcoreMesh(core_axis_name="c", subcore_axis_name="s",
                                  num_cores=NC, num_subcores=NS)
    f = pl.kernel(
        _sc_body,
        out_type=jax.ShapeDtypeStruct((N * N * D,), jnp.float32),
        mesh=mesh,
        scratch_types=[
            pltpu.VMEM((N,), jnp.float32),
            pltpu.VMEM((N,), jnp.float32),
            pltpu.VMEM((N + L,), jnp.float32),
            pltpu.VMEM((N + L,), jnp.float32),
            pltpu.VMEM((2 * D_PER_W,), jnp.int32),
            pltpu.VMEM((2 * D_PER_W, D), jnp.float32),
            pltpu.VMEM((NBUF, NB, D), jnp.float32),
            pltpu.SemaphoreType.DMA,
            pltpu.SemaphoreType.DMA,
            pltpu.SemaphoreType.DMA,
            pltpu.SemaphoreType.DMA,
        ],
        compiler_params=pltpu.CompilerParams(needs_layout_passes=False),
    )
    return f(x_col, y_col, a2d)


def kernel(xyz_sampled, alpha_volume):
    x_col = xyz_sampled[:, 0]
    y_col = xyz_sampled[:, 1]
    a2d = alpha_volume.reshape(R, D)
    return _alpha_grid(x_col, y_col, a2d)


# speedup vs baseline: 1.0312x; 1.0312x over previous
import functools

import jax
import jax.numpy as jnp
from jax import lax
from jax.experimental import pallas as pl
from jax.experimental.pallas import tpu as pltpu
from jax.experimental.pallas import tpu_sc as plsc

N = 64
R = 64
D = 64 * 64
NC = 2
NS = 16
L = 16
NW = NC * NS
D_PER_W = N // NW
NB = 8
NBATCH = N // NB
NBUF = 2


def _sc_body(x_hbm, y_hbm, a_hbm, out_hbm,
             x_v, y_v, w0_v, w1_v, idx_v, rows_v, out_v,
             sem_g, sem_o0, sem_o1, sem_o2):
    c = lax.axis_index("c")
    s = lax.axis_index("s")
    wid = s * NC + c
    d_base = wid * D_PER_W

    pltpu.sync_copy(x_hbm, x_v)
    pltpu.sync_copy(y_hbm, y_v)

    lanes = lax.iota(jnp.int32, L)

    samp = jnp.minimum(d_base + (lanes >> 1), N - 1)
    yv = plsc.load_gather(y_v, [samp])
    ty = yv * 2.0 - 1.0
    yy = 0.5 * ((ty + 1.0) * 62.0)
    y0i = yy.astype(jnp.int32)
    y0c = jnp.clip(y0i, 0, R - 1)
    y1c = jnp.clip(y0i + 1, 0, R - 1)
    rowidx = jnp.where((lanes & 1) == 0, y0c, y1c)
    plsc.store_scatter(idx_v, [lanes & 3], rowidx, mask=lanes < 2 * D_PER_W)

    gather = pltpu.async_copy(a_hbm.at[idx_v], rows_v, sem_g)

    for k in range(N // L):
        sl = pl.ds(k * L, L)
        xk = x_v[sl]
        yk = y_v[sl]
        txk = xk * 2.0 - 1.0
        xx = 0.5 * ((txk + 1.0) * 62.0)
        x0i = xx.astype(jnp.int32)
        x0f = jnp.clip(x0i, 0, R - 1).astype(jnp.float32)
        x1f = jnp.clip(x0i + 1, 0, R - 1).astype(jnp.float32)
        xfac = (x1f - xx) + (xx - x0f)
        tyk = yk * 2.0 - 1.0
        yyk = 0.5 * ((tyk + 1.0) * 62.0)
        ky0 = yyk.astype(jnp.int32)
        ky0f = jnp.clip(ky0, 0, R - 1).astype(jnp.float32)
        ky1f = jnp.clip(ky0 + 1, 0, R - 1).astype(jnp.float32)
        sl_w = pl.ds((k + 1) * L, L)
        w0_v[sl_w] = xfac * (ky1f - yyk)
        w1_v[sl_w] = xfac * (yyk - ky0f)

    gather.wait()

    out_handles = [[] for _ in range(NBUF)]
    out_sems = [sem_o0, sem_o1, sem_o2][:NBUF]
    for di in range(D_PER_W):
        d = d_base + di
        for nb in range(NBATCH):
            b = (di * NBATCH + nb) % NBUF
            w0b = [plsc.load_gather(w0_v, [jnp.full((L,), L + nb * NB + k,
                                                    jnp.int32)])
                   for k in range(NB)]
            w1b = [plsc.load_gather(w1_v, [jnp.full((L,), L + nb * NB + k,
                                                    jnp.int32)])
                   for k in range(NB)]
            for h in out_handles[b]:
                h.wait()
            out_handles[b] = []

            def chunk(j, di=di, b=b, w0b=w0b, w1b=w1b):
                sl = pl.ds(j * L, L)
                r0 = rows_v[2 * di, sl]
                r1 = rows_v[2 * di + 1, sl]
                for k in range(NB):
                    out_v[b, k, sl] = w0b[k] * r0 + w1b[k] * r1

            plsc.parallel_loop(0, D // L, 1, unroll=4)(chunk)
            out_handles[b] = [
                pltpu.async_copy(
                    out_v.at[b, k],
                    out_hbm.at[pl.ds((d * N + nb * NB + k) * D, D)],
                    out_sems[b])
                for k in range(NB)]
    for hs in out_handles:
        for h in hs:
            h.wait()


@jax.jit
def _alpha_grid(x_col, y_col, a2d):
    mesh = plsc.VectorSubcoreMesh(core_axis_name="c", subcore_axis_name="s",
                                  num_cores=NC, num_subcores=NS)
    f = pl.kernel(
        _sc_body,
        out_type=jax.ShapeDtypeStruct((N * N * D,), jnp.float32),
        mesh=mesh,
        scratch_types=[
            pltpu.VMEM((N,), jnp.float32),
            pltpu.VMEM((N,), jnp.float32),
            pltpu.VMEM((N + L,), jnp.float32),
            pltpu.VMEM((N + L,), jnp.float32),
            pltpu.VMEM((2 * D_PER_W,), jnp.int32),
            pltpu.VMEM((2 * D_PER_W, D), jnp.float32),
            pltpu.VMEM((NBUF, NB, D), jnp.float32),
            pltpu.SemaphoreType.DMA,
            pltpu.SemaphoreType.DMA,
            pltpu.SemaphoreType.DMA,
            pltpu.SemaphoreType.DMA,
        ],
        compiler_params=pltpu.CompilerParams(needs_layout_passes=False),
    )
    return f(x_col, y_col, a2d)


def kernel(xyz_sampled, alpha_volume):
    x_col = xyz_sampled[:, 0]
    y_col = xyz_sampled[:, 1]
    a2d = alpha_volume.reshape(R, D)
    return _alpha_grid(x_col, y_col, a2d)
